# baseline (speedup 1.0000x reference)
import functools

import jax
import jax.numpy as jnp
from jax.experimental import pallas as pl
from jax.experimental.pallas import tpu as pltpu

H = 128
EBLK = 2000
NBLK = 2000


def _celu(x):
    return jnp.where(x > 0, x, jnp.exp(jnp.minimum(x, 0.0)) - 1.0)


def _ln(x, g, beta):
    mu = jnp.mean(x, axis=-1, keepdims=True)
    xc = x - mu
    var = jnp.mean(xc * xc, axis=-1, keepdims=True)
    return xc * jax.lax.rsqrt(var + 1e-5) * g + beta


def _aux_pack(*rows):
    d = rows[0].shape[0]
    a = jnp.zeros((8, d), jnp.float32)
    for i, r in enumerate(rows):
        a = a.at[i].set(r)
    return a


def _enc_body(x_ref, w1_ref, w2_ref, aux_ref, out_ref, *, kdim):
    x = x_ref[...]
    acc = jnp.zeros((x.shape[0], H), jnp.float32) + aux_ref[0:1, :]
    for k in range(kdim):
        acc = acc + x[:, k:k + 1] * w1_ref[k:k + 1, :]
    t = _celu(acc)
    y = jnp.dot(t, w2_ref[...], preferred_element_type=jnp.float32)
    y = y + aux_ref[1:2, :]
    out_ref[...] = _ln(y, aux_ref[2:3, :], aux_ref[3:4, :])


def _pq_body(h_ref, w1b_ref, w1c_ref, p_ref, q_ref):
    h = h_ref[...]
    p_ref[...] = jnp.dot(h, w1b_ref[...], preferred_element_type=jnp.float32)
    q_ref[...] = jnp.dot(h, w1c_ref[...], preferred_element_type=jnp.float32)


def _edge_body(ea_ref, df_ref, sf_ref, w1a_ref, w2_ref, aux_ref,
               msg_ref, enew_ref):
    ea = ea_ref[...]
    t = jnp.dot(ea, w1a_ref[...], preferred_element_type=jnp.float32)
    t = t + df_ref[...] + sf_ref[...] + aux_ref[0:1, :]
    t = _celu(t)
    y = jnp.dot(t, w2_ref[...], preferred_element_type=jnp.float32)
    y = y + aux_ref[1:2, :]
    m = _ln(y, aux_ref[2:3, :], aux_ref[3:4, :])
    msg_ref[...] = m
    enew_ref[...] = ea + m


def _edge_last_body(ea_ref, df_ref, sf_ref, ef0_ref, eb0_ref,
                    w1a_ref, w2_ref, aux_ref,
                    msg_ref, enew_ref, ebwd_ref):
    ea = ea_ref[...]
    t = jnp.dot(ea, w1a_ref[...], preferred_element_type=jnp.float32)
    t = t + df_ref[...] + sf_ref[...] + aux_ref[0:1, :]
    t = _celu(t)
    y = jnp.dot(t, w2_ref[...], preferred_element_type=jnp.float32)
    y = y + aux_ref[1:2, :]
    m = _ln(y, aux_ref[2:3, :], aux_ref[3:4, :])
    enew = ea + m
    msg_ref[...] = m
    enew_ref[...] = enew
    ebwd_ref[...] = eb0_ref[...] + ef0_ref[...] - enew


def _node_body(h_ref, a0_ref, a1_ref, v1a_ref, v1b_ref, v2_ref, aux_ref,
               out_ref):
    h = h_ref[...]
    agg = a0_ref[...] - a1_ref[...]
    t = jnp.dot(h, v1a_ref[...], preferred_element_type=jnp.float32)
    t = t + jnp.dot(agg, v1b_ref[...], preferred_element_type=jnp.float32)
    t = _celu(t + aux_ref[0:1, :])
    y = jnp.dot(t, v2_ref[...], preferred_element_type=jnp.float32)
    y = y + aux_ref[1:2, :]
    out_ref[...] = h + _ln(y, aux_ref[2:3, :], aux_ref[3:4, :])


def _final_body(h_ref, i0_ref, i1_ref, d_ref, r_ref,
                lnaux_ref, acomb_ref, aaux_ref, bblk_ref, baux_ref,
                out_ref):
    inc = i0_ref[...] + i1_ref[...]
    z = jnp.concatenate([h_ref[...], inc], axis=1)
    z = _ln(z, lnaux_ref[0:1, :], lnaux_ref[1:2, :])
    t = jnp.dot(z, acomb_ref[...], preferred_element_type=jnp.float32)
    t = _celu(t + aaux_ref[0:1, :])
    y = jnp.dot(t, bblk_ref[...], preferred_element_type=jnp.float32)
    y = y + baux_ref[0:1, :]
    dm = 1.0 - d_ref[...]
    rm = 1.0 - r_ref[...]
    col = jax.lax.broadcasted_iota(jnp.int32, y.shape, 1)
    mask = jnp.where(col < 2, dm, jnp.where(col < 3, rm, 0.0))
    out_ref[...] = y * mask


def _full(shape):
    return pl.BlockSpec(shape, lambda i: (0, 0))


def _blk(m, pref):
    return pref if m % pref == 0 else m


def _rows(blk, width):
    return pl.BlockSpec((blk, width), lambda i: (i, 0))


def _enc_call(xp, w1, w2, aux, blk):
    m = xp.shape[0]
    kdim = w1.shape[0]
    return pl.pallas_call(
        functools.partial(_enc_body, kdim=kdim),
        grid=(m // blk,),
        in_specs=[_rows(blk, xp.shape[1]), _full(w1.shape), _full(w2.shape),
                  _full(aux.shape)],
        out_specs=_rows(blk, H),
        out_shape=jax.ShapeDtypeStruct((m, H), jnp.float32),
    )(xp, w1, w2, aux)


def _pq_call(h, w1b, w1c):
    n = h.shape[0]
    blk = _blk(n, NBLK)
    return pl.pallas_call(
        _pq_body,
        grid=(n // blk,),
        in_specs=[_rows(blk, H), _full((H, H)), _full((H, H))],
        out_specs=[_rows(blk, H), _rows(blk, H)],
        out_shape=[jax.ShapeDtypeStruct((n, H), jnp.float32),
                   jax.ShapeDtypeStruct((n, H), jnp.float32)],
    )(h, w1b, w1c)


def _edge_call(ea, df, sf, w1a, w2, aux):
    e = ea.shape[0]
    blk = _blk(e, EBLK)
    return pl.pallas_call(
        _edge_body,
        grid=(e // blk,),
        in_specs=[_rows(blk, H)] * 3 + [_full((H, H)), _full((H, H)),
                                        _full(aux.shape)],
        out_specs=[_rows(blk, H), _rows(blk, H)],
        out_shape=[jax.ShapeDtypeStruct((e, H), jnp.float32),
                   jax.ShapeDtypeStruct((e, H), jnp.float32)],
    )(ea, df, sf, w1a, w2, aux)


def _edge_last_call(ea, df, sf, ef0, eb0, w1a, w2, aux):
    e = ea.shape[0]
    blk = _blk(e, EBLK)
    return pl.pallas_call(
        _edge_last_body,
        grid=(e // blk,),
        in_specs=[_rows(blk, H)] * 5 + [_full((H, H)), _full((H, H)),
                                        _full(aux.shape)],
        out_specs=[_rows(blk, H)] * 3,
        out_shape=[jax.ShapeDtypeStruct((e, H), jnp.float32)] * 3,
    )(ea, df, sf, ef0, eb0, w1a, w2, aux)


def _node_call(h, a0, a1, v1a, v1b, v2, aux):
    n = h.shape[0]
    blk = _blk(n, NBLK)
    return pl.pallas_call(
        _node_body,
        grid=(n // blk,),
        in_specs=[_rows(blk, H)] * 3 + [_full((H, H))] * 3 + [_full(aux.shape)],
        out_specs=_rows(blk, H),
        out_shape=jax.ShapeDtypeStruct((n, H), jnp.float32),
    )(h, a0, a1, v1a, v1b, v2, aux)


def _final_call(h, i0, i1, d, r, lnaux, acomb, aaux, bblk, baux):
    n = h.shape[0]
    blk = _blk(n, NBLK)
    return pl.pallas_call(
        _final_body,
        grid=(n // blk,),
        in_specs=[_rows(blk, H)] * 3 + [_rows(blk, 1)] * 2 +
                 [_full(lnaux.shape), _full(acomb.shape), _full(aaux.shape),
                  _full(bblk.shape), _full(baux.shape)],
        out_specs=_rows(blk, H),
        out_shape=jax.ShapeDtypeStruct((n, H), jnp.float32),
    )(h, i0, i1, d, r, lnaux, acomb, aaux, bblk, baux)


def _gather_rows(p, q, idx_dst, idx_src):
    return jnp.take(p, idx_dst, axis=0), jnp.take(q, idx_src, axis=0)


def _scatter_two(n, rows0, idx0, rows1, idx1):
    a0 = jnp.zeros((n, rows0.shape[1]), jnp.float32).at[idx0].add(rows0)
    a1 = jnp.zeros((n, rows1.shape[1]), jnp.float32).at[idx1].add(rows1)
    return a0, a1


def kernel(x, edge_index, edge_attr, n_elements, bc_disp, bc_rot, params):
    n = x.shape[0]
    e2 = edge_attr.shape[0]
    e = e2 // 2

    idx_src = edge_index[0, :e].astype(jnp.int32)
    idx_dst = edge_index[1, :e].astype(jnp.int32)

    ne = params["node_enc"]
    (nw1, nb1), (nw2, nb2) = ne["layers"]
    ng, nbeta = ne["ln"]
    xp = jnp.pad(x, ((0, 0), (0, 16 - x.shape[1])))
    nw1p = jnp.pad(nw1, ((0, 16 - nw1.shape[0]), (0, 0)))
    h = _enc_call(xp, nw1p, nw2, _aux_pack(nb1, nb2, ng, nbeta),
                  _blk(x.shape[0], NBLK))

    ee = params["edge_enc"]
    (ew1, eb1), (ew2, eb2) = ee["layers"]
    eg, ebeta = ee["ln"]
    eap = jnp.pad(edge_attr, ((0, 0), (0, 8 - edge_attr.shape[1])))
    ew1p = jnp.pad(ew1, ((0, 8 - ew1.shape[0]), (0, 0)))
    e_all0 = _enc_call(eap, ew1p, ew2, _aux_pack(eb1, eb2, eg, ebeta),
                       _blk(e2, EBLK))
    e_fwd0 = e_all0[:e]
    e_bwd0 = e_all0[e:]

    e_fwd = e_fwd0
    n_layers = len(params["mp"])
    for li, layer in enumerate(params["mp"]):
        (w1, b1), (w2, b2) = layer["edge_mlp"]["layers"]
        g, beta = layer["edge_mlp"]["ln"]
        w1a, w1b, w1c = w1[:H], w1[H:2 * H], w1[2 * H:]
        eaux = _aux_pack(b1, b2, g, beta)

        p, q = _pq_call(h, w1b, w1c)
        df, sf = _gather_rows(p, q, idx_dst, idx_src)

        if li == n_layers - 1:
            msg, e_fwd_new, e_bwd_f = _edge_last_call(
                e_fwd, df, sf, e_fwd0, e_bwd0, w1a, w2, eaux)
        else:
            msg, e_fwd_new = _edge_call(e_fwd, df, sf, w1a, w2, eaux)
        e_fwd = e_fwd_new

        a0, a1 = _scatter_two(n, msg, idx_dst, msg, idx_src)

        (v1, c1), (v2, c2) = layer["node_mlp"]["layers"]
        vg, vbeta = layer["node_mlp"]["ln"]
        v1a, v1b = v1[:H], v1[H:]
        h = _node_call(h, a0, a1, v1a, v1b, v2,
                       _aux_pack(c1, c2, vg, vbeta))

    i0, i1 = _scatter_two(n, e_fwd, idx_dst, e_bwd_f, idx_src)

    fg, fbeta = params["final_ln"]
    lnaux = jnp.stack([fg, fbeta])
    lnaux = jnp.concatenate([lnaux, jnp.zeros((6, 2 * H), jnp.float32)])

    (aux_w1, aux_b1), (aux_w2, aux_b2) = params["dec_ux"]["layers"]
    (auz_w1, auz_b1), (auz_w2, auz_b2) = params["dec_uz"]["layers"]
    (ath_w1, ath_b1), (ath_w2, ath_b2) = params["dec_th"]["layers"]
    acomb = jnp.concatenate([aux_w1, auz_w1, ath_w1], axis=1)
    abias = jnp.concatenate([aux_b1, auz_b1, ath_b1])
    aaux = _aux_pack(abias)
    bblk = jnp.zeros((3 * H, H), jnp.float32)
    bblk = bblk.at[0:H, 0:1].set(aux_w2)
    bblk = bblk.at[H:2 * H, 1:2].set(auz_w2)
    bblk = bblk.at[2 * H:3 * H, 2:3].set(ath_w2)
    bbias = jnp.zeros((H,), jnp.float32)
    bbias = bbias.at[0].set(aux_b2[0]).at[1].set(auz_b2[0]).at[2].set(ath_b2[0])
    baux = _aux_pack(bbias)

    pred = _final_call(h, i0, i1, bc_disp, bc_rot,
                       lnaux, acomb, aaux, bblk, baux)
    return pred[:, :3]


# speedup vs baseline: 2.8024x; 2.8024x over previous
import functools

import jax
import jax.numpy as jnp
from jax import lax
from jax.experimental import pallas as pl
from jax.experimental.pallas import tpu as pltpu
from jax.experimental.pallas import tpu_sc as plsc

H = 128
EBLK = 2000
NBLK = 2000


def _celu(x):
    return jnp.where(x > 0, x, jnp.exp(jnp.minimum(x, 0.0)) - 1.0)


def _ln(x, g, beta):
    mu = jnp.mean(x, axis=-1, keepdims=True)
    xc = x - mu
    var = jnp.mean(xc * xc, axis=-1, keepdims=True)
    return xc * jax.lax.rsqrt(var + 1e-5) * g + beta


def _aux_pack(*rows):
    d = rows[0].shape[0]
    a = jnp.zeros((8, d), jnp.float32)
    for i, r in enumerate(rows):
        a = a.at[i].set(r)
    return a


def _enc_body(x_ref, w1_ref, w2_ref, aux_ref, out_ref, *, kdim):
    x = x_ref[...]
    acc = jnp.zeros((x.shape[0], H), jnp.float32) + aux_ref[0:1, :]
    for k in range(kdim):
        acc = acc + x[:, k:k + 1] * w1_ref[k:k + 1, :]
    t = _celu(acc)
    y = jnp.dot(t, w2_ref[...], preferred_element_type=jnp.float32)
    y = y + aux_ref[1:2, :]
    out_ref[...] = _ln(y, aux_ref[2:3, :], aux_ref[3:4, :])


def _pq_body(h_ref, w1b_ref, w1c_ref, p_ref, q_ref):
    h = h_ref[...]
    p_ref[...] = jnp.dot(h, w1b_ref[...], preferred_element_type=jnp.float32)
    q_ref[...] = jnp.dot(h, w1c_ref[...], preferred_element_type=jnp.float32)


def _edge_body(ea_ref, df_ref, sf_ref, w1a_ref, w2_ref, aux_ref,
               msg_ref, enew_ref):
    ea = ea_ref[...]
    t = jnp.dot(ea, w1a_ref[...], preferred_element_type=jnp.float32)
    t = t + df_ref[...] + sf_ref[...] + aux_ref[0:1, :]
    t = _celu(t)
    y = jnp.dot(t, w2_ref[...], preferred_element_type=jnp.float32)
    y = y + aux_ref[1:2, :]
    m = _ln(y, aux_ref[2:3, :], aux_ref[3:4, :])
    msg_ref[...] = m
    enew_ref[...] = ea + m


def _edge_last_body(ea_ref, df_ref, sf_ref, ef0_ref, eb0_ref,
                    w1a_ref, w2_ref, aux_ref,
                    msg_ref, enew_ref, ebwd_ref):
    ea = ea_ref[...]
    t = jnp.dot(ea, w1a_ref[...], preferred_element_type=jnp.float32)
    t = t + df_ref[...] + sf_ref[...] + aux_ref[0:1, :]
    t = _celu(t)
    y = jnp.dot(t, w2_ref[...], preferred_element_type=jnp.float32)
    y = y + aux_ref[1:2, :]
    m = _ln(y, aux_ref[2:3, :], aux_ref[3:4, :])
    enew = ea + m
    msg_ref[...] = m
    enew_ref[...] = enew
    ebwd_ref[...] = eb0_ref[...] + ef0_ref[...] - enew


def _node_body(h_ref, a0_ref, a1_ref, v1a_ref, v1b_ref, v2_ref, aux_ref,
               out_ref):
    h = h_ref[...]
    agg = a0_ref[...] - a1_ref[...]
    t = jnp.dot(h, v1a_ref[...], preferred_element_type=jnp.float32)
    t = t + jnp.dot(agg, v1b_ref[...], preferred_element_type=jnp.float32)
    t = _celu(t + aux_ref[0:1, :])
    y = jnp.dot(t, v2_ref[...], preferred_element_type=jnp.float32)
    y = y + aux_ref[1:2, :]
    out_ref[...] = h + _ln(y, aux_ref[2:3, :], aux_ref[3:4, :])


def _final_body(h_ref, i0_ref, i1_ref, d_ref, r_ref,
                lnaux_ref, acomb_ref, aaux_ref, bblk_ref, baux_ref,
                out_ref):
    inc = i0_ref[...] + i1_ref[...]
    z = jnp.concatenate([h_ref[...], inc], axis=1)
    z = _ln(z, lnaux_ref[0:1, :], lnaux_ref[1:2, :])
    t = jnp.dot(z, acomb_ref[...], preferred_element_type=jnp.float32)
    t = _celu(t + aaux_ref[0:1, :])
    y = jnp.dot(t, bblk_ref[...], preferred_element_type=jnp.float32)
    y = y + baux_ref[0:1, :]
    dm = 1.0 - d_ref[...]
    rm = 1.0 - r_ref[...]
    col = jax.lax.broadcasted_iota(jnp.int32, y.shape, 1)
    mask = jnp.where(col < 2, dm, jnp.where(col < 3, rm, 0.0))
    out_ref[...] = y * mask


def _full(shape):
    return pl.BlockSpec(shape, lambda i: (0, 0))


def _blk(m, pref):
    return pref if m % pref == 0 else m


def _rows(blk, width):
    return pl.BlockSpec((blk, width), lambda i: (i, 0))


def _enc_call(xp, w1, w2, aux, blk):
    m = xp.shape[0]
    kdim = w1.shape[0]
    return pl.pallas_call(
        functools.partial(_enc_body, kdim=kdim),
        grid=(m // blk,),
        in_specs=[_rows(blk, xp.shape[1]), _full(w1.shape), _full(w2.shape),
                  _full(aux.shape)],
        out_specs=_rows(blk, H),
        out_shape=jax.ShapeDtypeStruct((m, H), jnp.float32),
    )(xp, w1, w2, aux)


def _pq_call(h, w1b, w1c):
    n = h.shape[0]
    blk = _blk(n, NBLK)
    return pl.pallas_call(
        _pq_body,
        grid=(n // blk,),
        in_specs=[_rows(blk, H), _full((H, H)), _full((H, H))],
        out_specs=[_rows(blk, H), _rows(blk, H)],
        out_shape=[jax.ShapeDtypeStruct((n, H), jnp.float32),
                   jax.ShapeDtypeStruct((n, H), jnp.float32)],
    )(h, w1b, w1c)


def _edge_call(ea, df, sf, w1a, w2, aux):
    e = ea.shape[0]
    blk = _blk(e, EBLK)
    return pl.pallas_call(
        _edge_body,
        grid=(e // blk,),
        in_specs=[_rows(blk, H)] * 3 + [_full((H, H)), _full((H, H)),
                                        _full(aux.shape)],
        out_specs=[_rows(blk, H), _rows(blk, H)],
        out_shape=[jax.ShapeDtypeStruct((e, H), jnp.float32),
                   jax.ShapeDtypeStruct((e, H), jnp.float32)],
    )(ea, df, sf, w1a, w2, aux)


def _edge_last_call(ea, df, sf, ef0, eb0, w1a, w2, aux):
    e = ea.shape[0]
    blk = _blk(e, EBLK)
    return pl.pallas_call(
        _edge_last_body,
        grid=(e // blk,),
        in_specs=[_rows(blk, H)] * 5 + [_full((H, H)), _full((H, H)),
                                        _full(aux.shape)],
        out_specs=[_rows(blk, H)] * 3,
        out_shape=[jax.ShapeDtypeStruct((e, H), jnp.float32)] * 3,
    )(ea, df, sf, ef0, eb0, w1a, w2, aux)


def _node_call(h, a0, a1, v1a, v1b, v2, aux):
    n = h.shape[0]
    blk = _blk(n, NBLK)
    return pl.pallas_call(
        _node_body,
        grid=(n // blk,),
        in_specs=[_rows(blk, H)] * 3 + [_full((H, H))] * 3 + [_full(aux.shape)],
        out_specs=_rows(blk, H),
        out_shape=jax.ShapeDtypeStruct((n, H), jnp.float32),
    )(h, a0, a1, v1a, v1b, v2, aux)


def _final_call(h, i0, i1, d, r, lnaux, acomb, aaux, bblk, baux):
    n = h.shape[0]
    blk = _blk(n, NBLK)
    return pl.pallas_call(
        _final_body,
        grid=(n // blk,),
        in_specs=[_rows(blk, H)] * 3 + [_rows(blk, 1)] * 2 +
                 [_full(lnaux.shape), _full(acomb.shape), _full(aaux.shape),
                  _full(bblk.shape), _full(baux.shape)],
        out_specs=_rows(blk, H),
        out_shape=jax.ShapeDtypeStruct((n, H), jnp.float32),
    )(h, i0, i1, d, r, lnaux, acomb, aaux, bblk, baux)


_NC = 2
_NS = 16
_CHUNK = 128


def _sc_gather(p, q, idxd, idxs):
    e = idxd.shape[0]
    nchunk = e // _CHUNK
    nw = _NC * _NS
    per_w = (nchunk + nw - 1) // nw
    mesh = plsc.VectorSubcoreMesh(core_axis_name="c", subcore_axis_name="s")

    @functools.partial(
        pl.kernel, mesh=mesh,
        out_type=[jax.ShapeDtypeStruct((e, H), jnp.float32),
                  jax.ShapeDtypeStruct((e, H), jnp.float32)],
        scratch_types=[pltpu.VMEM((_CHUNK,), jnp.int32),
                       pltpu.VMEM((_CHUNK,), jnp.int32),
                       pltpu.VMEM((_CHUNK, H), jnp.float32),
                       pltpu.VMEM((_CHUNK, H), jnp.float32),
                       pltpu.SemaphoreType.DMA,
                       pltpu.SemaphoreType.DMA],
    )
    def k(p_hbm, q_hbm, idxd_hbm, idxs_hbm, df_hbm, sf_hbm,
          idxd_v, idxs_v, rowd_v, rows_v, sem1, sem2):
        wid = lax.axis_index("s") * _NC + lax.axis_index("c")

        def body(i, carry):
            cid = wid + i * nw

            @pl.when(cid < nchunk)
            def _():
                base = cid * _CHUNK
                pltpu.sync_copy(idxd_hbm.at[pl.ds(base, _CHUNK)], idxd_v)
                pltpu.sync_copy(idxs_hbm.at[pl.ds(base, _CHUNK)], idxs_v)
                c1 = pltpu.async_copy(p_hbm.at[idxd_v], rowd_v, sem1)
                c2 = pltpu.async_copy(q_hbm.at[idxs_v], rows_v, sem2)
                c1.wait()
                c2.wait()
                pltpu.sync_copy(rowd_v, df_hbm.at[pl.ds(base, _CHUNK)])
                pltpu.sync_copy(rows_v, sf_hbm.at[pl.ds(base, _CHUNK)])

            return carry

        lax.fori_loop(0, per_w, body, 0)

    return k(p, q, idxd, idxs)


def _sc_scatter(n, rows0, idx0, rows1, idx1, zeros):
    e = idx0.shape[0]
    nchunk = e // _CHUNK
    per_t = (nchunk + _NS - 1) // _NS
    rpt = -(-n // (8 * _NS)) * 8
    acc_n = rpt * _NS
    tail = n - rpt * (_NS - 1)
    mesh = plsc.VectorSubcoreMesh(core_axis_name="c", subcore_axis_name="s")

    @functools.partial(
        pl.kernel, mesh=mesh,
        out_type=jax.ShapeDtypeStruct((_NC, n, H), jnp.float32),
        scratch_types=[pltpu.VMEM((_CHUNK,), jnp.int32),
                       pltpu.VMEM((_CHUNK, H), jnp.float32),
                       pltpu.VMEM_SHARED((acc_n, H), jnp.float32)],
    )
    def k(r0_hbm, r1_hbm, idx0_hbm, idx1_hbm, z_hbm, out_hbm,
          idx_v, rows_v, acc_sh):
        c = lax.axis_index("c")
        s = lax.axis_index("s")
        rbase = s * rpt

        @pl.when(s < _NS - 1)
        def _():
            pltpu.sync_copy(z_hbm.at[pl.ds(rbase, rpt)],
                            acc_sh.at[pl.ds(rbase, rpt)])

        @pl.when(s == _NS - 1)
        def _():
            pltpu.sync_copy(z_hbm.at[pl.ds((_NS - 1) * rpt, tail)],
                            acc_sh.at[pl.ds((_NS - 1) * rpt, tail)])

        plsc.subcore_barrier()

        def body(i, carry):
            cid = s + i * _NS

            @pl.when(cid < nchunk)
            def _():
                base = cid * _CHUNK

                @pl.when(c == 0)
                def _():
                    pltpu.sync_copy(idx0_hbm.at[pl.ds(base, _CHUNK)], idx_v)
                    pltpu.sync_copy(r0_hbm.at[pl.ds(base, _CHUNK)], rows_v)

                @pl.when(c == 1)
                def _():
                    pltpu.sync_copy(idx1_hbm.at[pl.ds(base, _CHUNK)], idx_v)
                    pltpu.sync_copy(r1_hbm.at[pl.ds(base, _CHUNK)], rows_v)

                pltpu.sync_copy(rows_v, acc_sh.at[idx_v], add=True)

            return carry

        lax.fori_loop(0, per_t, body, 0)
        plsc.subcore_barrier()

        @pl.when(s < _NS - 1)
        def _():
            pltpu.sync_copy(acc_sh.at[pl.ds(rbase, rpt)],
                            out_hbm.at[c, pl.ds(rbase, rpt)])

        @pl.when(s == _NS - 1)
        def _():
            pltpu.sync_copy(acc_sh.at[pl.ds((_NS - 1) * rpt, tail)],
                            out_hbm.at[c, pl.ds((_NS - 1) * rpt, tail)])

    return k(rows0, rows1, idx0, idx1, zeros)


def kernel(x, edge_index, edge_attr, n_elements, bc_disp, bc_rot, params):
    n = x.shape[0]
    e2 = edge_attr.shape[0]
    e = e2 // 2

    idx_src = edge_index[0, :e].astype(jnp.int32)
    idx_dst = edge_index[1, :e].astype(jnp.int32)
    zeros = jnp.zeros((n, H), jnp.float32)

    ne = params["node_enc"]
    (nw1, nb1), (nw2, nb2) = ne["layers"]
    ng, nbeta = ne["ln"]
    xp = jnp.pad(x, ((0, 0), (0, 16 - x.shape[1])))
    nw1p = jnp.pad(nw1, ((0, 16 - nw1.shape[0]), (0, 0)))
    h = _enc_call(xp, nw1p, nw2, _aux_pack(nb1, nb2, ng, nbeta),
                  _blk(x.shape[0], NBLK))

    ee = params["edge_enc"]
    (ew1, eb1), (ew2, eb2) = ee["layers"]
    eg, ebeta = ee["ln"]
    eap = jnp.pad(edge_attr, ((0, 0), (0, 8 - edge_attr.shape[1])))
    ew1p = jnp.pad(ew1, ((0, 8 - ew1.shape[0]), (0, 0)))
    e_all0 = _enc_call(eap, ew1p, ew2, _aux_pack(eb1, eb2, eg, ebeta),
                       _blk(e2, EBLK))
    e_fwd0 = e_all0[:e]
    e_bwd0 = e_all0[e:]

    e_fwd = e_fwd0
    n_layers = len(params["mp"])
    for li, layer in enumerate(params["mp"]):
        (w1, b1), (w2, b2) = layer["edge_mlp"]["layers"]
        g, beta = layer["edge_mlp"]["ln"]
        w1a, w1b, w1c = w1[:H], w1[H:2 * H], w1[2 * H:]
        eaux = _aux_pack(b1, b2, g, beta)

        p, q = _pq_call(h, w1b, w1c)
        df, sf = _sc_gather(p, q, idx_dst, idx_src)

        if li == n_layers - 1:
            msg, e_fwd_new, e_bwd_f = _edge_last_call(
                e_fwd, df, sf, e_fwd0, e_bwd0, w1a, w2, eaux)
        else:
            msg, e_fwd_new = _edge_call(e_fwd, df, sf, w1a, w2, eaux)
        e_fwd = e_fwd_new

        parts = _sc_scatter(n, msg, idx_dst, msg, idx_src, zeros)
        a0, a1 = parts[0], parts[1]

        (v1, c1), (v2, c2) = layer["node_mlp"]["layers"]
        vg, vbeta = layer["node_mlp"]["ln"]
        v1a, v1b = v1[:H], v1[H:]
        h = _node_call(h, a0, a1, v1a, v1b, v2,
                       _aux_pack(c1, c2, vg, vbeta))

    fparts = _sc_scatter(n, e_fwd, idx_dst, e_bwd_f, idx_src, zeros)
    i0, i1 = fparts[0], fparts[1]

    fg, fbeta = params["final_ln"]
    lnaux = jnp.stack([fg, fbeta])
    lnaux = jnp.concatenate([lnaux, jnp.zeros((6, 2 * H), jnp.float32)])

    (aux_w1, aux_b1), (aux_w2, aux_b2) = params["dec_ux"]["layers"]
    (auz_w1, auz_b1), (auz_w2, auz_b2) = params["dec_uz"]["layers"]
    (ath_w1, ath_b1), (ath_w2, ath_b2) = params["dec_th"]["layers"]
    acomb = jnp.concatenate([aux_w1, auz_w1, ath_w1], axis=1)
    abias = jnp.concatenate([aux_b1, auz_b1, ath_b1])
    aaux = _aux_pack(abias)
    bblk = jnp.zeros((3 * H, H), jnp.float32)
    bblk = bblk.at[0:H, 0:1].set(aux_w2)
    bblk = bblk.at[H:2 * H, 1:2].set(auz_w2)
    bblk = bblk.at[2 * H:3 * H, 2:3].set(ath_w2)
    bbias = jnp.zeros((H,), jnp.float32)
    bbias = bbias.at[0].set(aux_b2[0]).at[1].set(auz_b2[0]).at[2].set(ath_b2[0])
    baux = _aux_pack(bbias)

    pred = _final_call(h, i0, i1, bc_disp, bc_rot,
                       lnaux, acomb, aaux, bblk, baux)
    return pred[:, :3]


# speedup vs baseline: 3.4614x; 1.2352x over previous
import functools

import jax
import jax.numpy as jnp
from jax import lax
from jax.experimental import pallas as pl
from jax.experimental.pallas import tpu as pltpu
from jax.experimental.pallas import tpu_sc as plsc

H = 128
EBLK = 2000
NBLK = 2000


def _celu(x):
    return jnp.where(x > 0, x, jnp.exp(jnp.minimum(x, 0.0)) - 1.0)


def _ln(x, g, beta):
    mu = jnp.mean(x, axis=-1, keepdims=True)
    xc = x - mu
    var = jnp.mean(xc * xc, axis=-1, keepdims=True)
    return xc * jax.lax.rsqrt(var + 1e-5) * g + beta


def _aux_pack(*rows):
    d = rows[0].shape[0]
    a = jnp.zeros((8, d), jnp.float32)
    for i, r in enumerate(rows):
        a = a.at[i].set(r)
    return a


def _enc_body(x_ref, w1_ref, w2_ref, aux_ref, out_ref, *, kdim):
    x = x_ref[...]
    acc = jnp.zeros((x.shape[0], H), jnp.float32) + aux_ref[0:1, :]
    for k in range(kdim):
        acc = acc + x[:, k:k + 1] * w1_ref[k:k + 1, :]
    t = _celu(acc)
    y = jnp.dot(t, w2_ref[...], preferred_element_type=jnp.float32)
    y = y + aux_ref[1:2, :]
    out_ref[...] = _ln(y, aux_ref[2:3, :], aux_ref[3:4, :])


def _pq_body(h_ref, w1b_ref, w1c_ref, p_ref, q_ref):
    h = h_ref[...]
    p_ref[...] = jnp.dot(h, w1b_ref[...], preferred_element_type=jnp.float32)
    q_ref[...] = jnp.dot(h, w1c_ref[...], preferred_element_type=jnp.float32)


def _edge_body(ea_ref, r_ref, w1a_ref, w2_ref, aux_ref,
               msg_ref, enew_ref):
    ea = ea_ref[...]
    t = jnp.dot(ea, w1a_ref[...], preferred_element_type=jnp.float32)
    t = t + r_ref[...] + aux_ref[0:1, :]
    t = _celu(t)
    y = jnp.dot(t, w2_ref[...], preferred_element_type=jnp.float32)
    y = y + aux_ref[1:2, :]
    m = _ln(y, aux_ref[2:3, :], aux_ref[3:4, :])
    msg_ref[...] = m
    enew_ref[...] = ea + m


def _edge_last_body(ea_ref, r_ref, ef0_ref, eb0_ref,
                    w1a_ref, w2_ref, aux_ref,
                    msg_ref, enew_ref, ebwd_ref):
    ea = ea_ref[...]
    t = jnp.dot(ea, w1a_ref[...], preferred_element_type=jnp.float32)
    t = t + r_ref[...] + aux_ref[0:1, :]
    t = _celu(t)
    y = jnp.dot(t, w2_ref[...], preferred_element_type=jnp.float32)
    y = y + aux_ref[1:2, :]
    m = _ln(y, aux_ref[2:3, :], aux_ref[3:4, :])
    enew = ea + m
    msg_ref[...] = m
    enew_ref[...] = enew
    ebwd_ref[...] = eb0_ref[...] + ef0_ref[...] - enew


def _node_body(h_ref, a0_ref, a1_ref, v1a_ref, v1b_ref, v2_ref, aux_ref,
               out_ref):
    h = h_ref[...]
    agg = a0_ref[...] - a1_ref[...]
    t = jnp.dot(h, v1a_ref[...], preferred_element_type=jnp.float32)
    t = t + jnp.dot(agg, v1b_ref[...], preferred_element_type=jnp.float32)
    t = _celu(t + aux_ref[0:1, :])
    y = jnp.dot(t, v2_ref[...], preferred_element_type=jnp.float32)
    y = y + aux_ref[1:2, :]
    out_ref[...] = h + _ln(y, aux_ref[2:3, :], aux_ref[3:4, :])


def _final_body(h_ref, i0_ref, i1_ref, d_ref, r_ref,
                lnaux_ref, acomb_ref, aaux_ref, bblk_ref, baux_ref,
                out_ref):
    inc = i0_ref[...] + i1_ref[...]
    z = jnp.concatenate([h_ref[...], inc], axis=1)
    z = _ln(z, lnaux_ref[0:1, :], lnaux_ref[1:2, :])
    t = jnp.dot(z, acomb_ref[...], preferred_element_type=jnp.float32)
    t = _celu(t + aaux_ref[0:1, :])
    y = jnp.dot(t, bblk_ref[...], preferred_element_type=jnp.float32)
    y = y + baux_ref[0:1, :]
    dm = 1.0 - d_ref[...]
    rm = 1.0 - r_ref[...]
    col = jax.lax.broadcasted_iota(jnp.int32, y.shape, 1)
    mask = jnp.where(col < 2, dm, jnp.where(col < 3, rm, 0.0))
    out_ref[...] = y * mask


def _full(shape):
    return pl.BlockSpec(shape, lambda i: (0, 0))


def _blk(m, pref):
    return pref if m % pref == 0 else m


def _rows(blk, width):
    return pl.BlockSpec((blk, width), lambda i: (i, 0))


def _enc_call(xp, w1, w2, aux, blk):
    m = xp.shape[0]
    kdim = w1.shape[0]
    return pl.pallas_call(
        functools.partial(_enc_body, kdim=kdim),
        grid=(m // blk,),
        in_specs=[_rows(blk, xp.shape[1]), _full(w1.shape), _full(w2.shape),
                  _full(aux.shape)],
        out_specs=_rows(blk, H),
        out_shape=jax.ShapeDtypeStruct((m, H), jnp.float32),
    )(xp, w1, w2, aux)


def _pq_call(h, w1b, w1c):
    n = h.shape[0]
    blk = _blk(n, NBLK)
    return pl.pallas_call(
        _pq_body,
        grid=(n // blk,),
        in_specs=[_rows(blk, H), _full((H, H)), _full((H, H))],
        out_specs=[_rows(blk, H), _rows(blk, H)],
        out_shape=[jax.ShapeDtypeStruct((n, H), jnp.float32),
                   jax.ShapeDtypeStruct((n, H), jnp.float32)],
    )(h, w1b, w1c)


def _edge_call(ea, r, w1a, w2, aux):
    e = ea.shape[0]
    blk = _blk(e, EBLK)
    return pl.pallas_call(
        _edge_body,
        grid=(e // blk,),
        in_specs=[_rows(blk, H)] * 2 + [_full((H, H)), _full((H, H)),
                                        _full(aux.shape)],
        out_specs=[_rows(blk, H), _rows(blk, H)],
        out_shape=[jax.ShapeDtypeStruct((e, H), jnp.float32),
                   jax.ShapeDtypeStruct((e, H), jnp.float32)],
    )(ea, r, w1a, w2, aux)


def _edge_last_call(ea, r, ef0, eb0, w1a, w2, aux):
    e = ea.shape[0]
    blk = _blk(e, EBLK)
    return pl.pallas_call(
        _edge_last_body,
        grid=(e // blk,),
        in_specs=[_rows(blk, H)] * 4 + [_full((H, H)), _full((H, H)),
                                        _full(aux.shape)],
        out_specs=[_rows(blk, H)] * 3,
        out_shape=[jax.ShapeDtypeStruct((e, H), jnp.float32)] * 3,
    )(ea, r, ef0, eb0, w1a, w2, aux)


def _node_call(h, a0, a1, v1a, v1b, v2, aux):
    n = h.shape[0]
    blk = _blk(n, NBLK)
    return pl.pallas_call(
        _node_body,
        grid=(n // blk,),
        in_specs=[_rows(blk, H)] * 3 + [_full((H, H))] * 3 + [_full(aux.shape)],
        out_specs=_rows(blk, H),
        out_shape=jax.ShapeDtypeStruct((n, H), jnp.float32),
    )(h, a0, a1, v1a, v1b, v2, aux)


def _final_call(h, i0, i1, d, r, lnaux, acomb, aaux, bblk, baux):
    n = h.shape[0]
    blk = _blk(n, NBLK)
    return pl.pallas_call(
        _final_body,
        grid=(n // blk,),
        in_specs=[_rows(blk, H)] * 3 + [_rows(blk, 1)] * 2 +
                 [_full(lnaux.shape), _full(acomb.shape), _full(aaux.shape),
                  _full(bblk.shape), _full(baux.shape)],
        out_specs=_rows(blk, H),
        out_shape=jax.ShapeDtypeStruct((n, H), jnp.float32),
    )(h, i0, i1, d, r, lnaux, acomb, aaux, bblk, baux)


_NC = 2
_NS = 16
_CHUNK = 128


_GG = 6
_GS = 3


def _sc_gather(p, q, idxd2, idxs2, e):
    nchunk = e // _CHUNK
    nsc = -(-nchunk // _GG)
    nw = _NC * _NS
    per_w = -(-nsc // nw)
    mesh = plsc.VectorSubcoreMesh(core_axis_name="c", subcore_axis_name="s")

    @functools.partial(
        pl.kernel, mesh=mesh,
        out_type=jax.ShapeDtypeStruct((e, H), jnp.float32),
        scratch_types=[pltpu.VMEM((_GG, _CHUNK), jnp.int32),
                       pltpu.VMEM((_GG, _CHUNK), jnp.int32),
                       pltpu.VMEM((_GG * _CHUNK, H), jnp.float32),
                       pltpu.SemaphoreType.DMA,
                       pltpu.SemaphoreType.DMA,
                       pltpu.SemaphoreType.DMA],
    )
    def k(p_hbm, q_hbm, idxd_hbm, idxs_hbm, r_hbm,
          idxd_v, idxs_v, rows_v, sem1, sem2, sem3):
        wid = lax.axis_index("s") * _NC + lax.axis_index("c")

        def body(t, carry):
            sc = t * nw + wid

            @pl.when(sc < nsc)
            def _():
                pltpu.sync_copy(idxd_hbm.at[sc], idxd_v)
                pltpu.sync_copy(idxs_hbm.at[sc], idxs_v)
                for j in range(_GG):
                    @pl.when(sc * _GG + j < nchunk)
                    def _(j=j):
                        pltpu.async_copy(
                            p_hbm.at[idxd_v.at[j]],
                            rows_v.at[pl.ds(j * _CHUNK, _CHUNK)], sem1)
                for j in range(_GG):
                    @pl.when(sc * _GG + j < nchunk)
                    def _(j=j):
                        pltpu.make_async_copy(
                            p_hbm.at[idxd_v.at[j]],
                            rows_v.at[pl.ds(j * _CHUNK, _CHUNK)], sem1).wait()
                for j in range(_GG):
                    @pl.when(sc * _GG + j < nchunk)
                    def _(j=j):
                        pltpu.async_copy(
                            q_hbm.at[idxs_v.at[j]],
                            rows_v.at[pl.ds(j * _CHUNK, _CHUNK)], sem2,
                            add=True)
                for j in range(_GG):
                    @pl.when(sc * _GG + j < nchunk)
                    def _(j=j):
                        pltpu.make_async_copy(
                            q_hbm.at[idxs_v.at[j]],
                            rows_v.at[pl.ds(j * _CHUNK, _CHUNK)], sem2).wait()
                for j in range(_GG):
                    @pl.when(sc * _GG + j < nchunk)
                    def _(j=j):
                        pltpu.async_copy(
                            rows_v.at[pl.ds(j * _CHUNK, _CHUNK)],
                            r_hbm.at[pl.ds((sc * _GG + j) * _CHUNK, _CHUNK)],
                            sem3)
                for j in range(_GG):
                    @pl.when(sc * _GG + j < nchunk)
                    def _(j=j):
                        pltpu.make_async_copy(
                            rows_v.at[pl.ds(j * _CHUNK, _CHUNK)],
                            r_hbm.at[pl.ds((sc * _GG + j) * _CHUNK, _CHUNK)],
                            sem3).wait()

            return carry

        lax.fori_loop(0, per_w, body, 0)

    return k(p, q, idxd2, idxs2)


def _sc_scatter(n, rows0, idx0_2, rows1, idx1_2, zeros, e):
    nchunk = e // _CHUNK
    nsc = -(-nchunk // _GS)
    per_t = -(-nsc // _NS)
    rpt = -(-n // (8 * _NS)) * 8
    acc_n = rpt * _NS
    tail = n - rpt * (_NS - 1)
    mesh = plsc.VectorSubcoreMesh(core_axis_name="c", subcore_axis_name="s")

    @functools.partial(
        pl.kernel, mesh=mesh,
        out_type=jax.ShapeDtypeStruct((_NC, n, H), jnp.float32),
        scratch_types=[pltpu.VMEM((_GS, _CHUNK), jnp.int32),
                       pltpu.VMEM((_GS * _CHUNK, H), jnp.float32),
                       pltpu.VMEM_SHARED((acc_n, H), jnp.float32),
                       pltpu.SemaphoreType.DMA,
                       pltpu.SemaphoreType.DMA],
    )
    def k(r0_hbm, r1_hbm, i0_hbm, i1_hbm, z_hbm, out_hbm,
          idx_v, rows_v, acc_sh, semA, semB):
        c = lax.axis_index("c")
        s = lax.axis_index("s")
        rbase = s * rpt

        @pl.when(s < _NS - 1)
        def _():
            pltpu.sync_copy(z_hbm.at[pl.ds(rbase, rpt)],
                            acc_sh.at[pl.ds(rbase, rpt)])

        @pl.when(s == _NS - 1)
        def _():
            pltpu.sync_copy(z_hbm.at[pl.ds((_NS - 1) * rpt, tail)],
                            acc_sh.at[pl.ds((_NS - 1) * rpt, tail)])

        plsc.subcore_barrier()

        def body(t, carry):
            sc = t * _NS + s

            @pl.when(sc < nsc)
            def _():
                @pl.when(c == 0)
                def _():
                    pltpu.sync_copy(i0_hbm.at[sc], idx_v)
                    for j in range(_GS):
                        @pl.when(sc * _GS + j < nchunk)
                        def _(j=j):
                            pltpu.async_copy(
                                r0_hbm.at[pl.ds((sc * _GS + j) * _CHUNK, _CHUNK)],
                                rows_v.at[pl.ds(j * _CHUNK, _CHUNK)], semA)

                @pl.when(c == 1)
                def _():
                    pltpu.sync_copy(i1_hbm.at[sc], idx_v)
                    for j in range(_GS):
                        @pl.when(sc * _GS + j < nchunk)
                        def _(j=j):
                            pltpu.async_copy(
                                r1_hbm.at[pl.ds((sc * _GS + j) * _CHUNK, _CHUNK)],
                                rows_v.at[pl.ds(j * _CHUNK, _CHUNK)], semA)

                for j in range(_GS):
                    @pl.when(sc * _GS + j < nchunk)
                    def _(j=j):
                        pltpu.make_async_copy(
                            r0_hbm.at[pl.ds((sc * _GS + j) * _CHUNK, _CHUNK)],
                            rows_v.at[pl.ds(j * _CHUNK, _CHUNK)], semA).wait()
                for j in range(_GS):
                    @pl.when(sc * _GS + j < nchunk)
                    def _(j=j):
                        pltpu.async_copy(
                            rows_v.at[pl.ds(j * _CHUNK, _CHUNK)],
                            acc_sh.at[idx_v.at[j]], semB, add=True)
                for j in range(_GS):
                    @pl.when(sc * _GS + j < nchunk)
                    def _(j=j):
                        pltpu.make_async_copy(
                            rows_v.at[pl.ds(j * _CHUNK, _CHUNK)],
                            acc_sh.at[idx_v.at[j]], semB).wait()

            return carry

        lax.fori_loop(0, per_t, body, 0)
        plsc.subcore_barrier()

        @pl.when(s < _NS - 1)
        def _():
            pltpu.sync_copy(acc_sh.at[pl.ds(rbase, rpt)],
                            out_hbm.at[c, pl.ds(rbase, rpt)])

        @pl.when(s == _NS - 1)
        def _():
            pltpu.sync_copy(acc_sh.at[pl.ds((_NS - 1) * rpt, tail)],
                            out_hbm.at[c, pl.ds((_NS - 1) * rpt, tail)])

    return k(rows0, rows1, idx0_2, idx1_2, zeros)


def kernel(x, edge_index, edge_attr, n_elements, bc_disp, bc_rot, params):
    n = x.shape[0]
    e2 = edge_attr.shape[0]
    e = e2 // 2

    idx_src = edge_index[0, :e].astype(jnp.int32)
    idx_dst = edge_index[1, :e].astype(jnp.int32)
    zeros = jnp.zeros((n, H), jnp.float32)
    nchunk = e // _CHUNK

    def _pad3(idx, g):
        nsc = -(-nchunk // g)
        return jnp.pad(idx.reshape(nchunk, _CHUNK),
                       ((0, nsc * g - nchunk), (0, 0))).reshape(nsc, g, _CHUNK)

    idx_dst_g = _pad3(idx_dst, _GG)
    idx_src_g = _pad3(idx_src, _GG)
    idx_dst_s = _pad3(idx_dst, _GS)
    idx_src_s = _pad3(idx_src, _GS)

    ne = params["node_enc"]
    (nw1, nb1), (nw2, nb2) = ne["layers"]
    ng, nbeta = ne["ln"]
    xp = jnp.pad(x, ((0, 0), (0, 16 - x.shape[1])))
    nw1p = jnp.pad(nw1, ((0, 16 - nw1.shape[0]), (0, 0)))
    h = _enc_call(xp, nw1p, nw2, _aux_pack(nb1, nb2, ng, nbeta),
                  _blk(x.shape[0], NBLK))

    ee = params["edge_enc"]
    (ew1, eb1), (ew2, eb2) = ee["layers"]
    eg, ebeta = ee["ln"]
    eap = jnp.pad(edge_attr, ((0, 0), (0, 8 - edge_attr.shape[1])))
    ew1p = jnp.pad(ew1, ((0, 8 - ew1.shape[0]), (0, 0)))
    e_all0 = _enc_call(eap, ew1p, ew2, _aux_pack(eb1, eb2, eg, ebeta),
                       _blk(e2, EBLK))
    e_fwd0 = e_all0[:e]
    e_bwd0 = e_all0[e:]

    e_fwd = e_fwd0
    n_layers = len(params["mp"])
    for li, layer in enumerate(params["mp"]):
        (w1, b1), (w2, b2) = layer["edge_mlp"]["layers"]
        g, beta = layer["edge_mlp"]["ln"]
        w1a, w1b, w1c = w1[:H], w1[H:2 * H], w1[2 * H:]
        eaux = _aux_pack(b1, b2, g, beta)

        p, q = _pq_call(h, w1b, w1c)
        r = _sc_gather(p, q, idx_dst_g, idx_src_g, e)

        if li == n_layers - 1:
            msg, e_fwd_new, e_bwd_f = _edge_last_call(
                e_fwd, r, e_fwd0, e_bwd0, w1a, w2, eaux)
        else:
            msg, e_fwd_new = _edge_call(e_fwd, r, w1a, w2, eaux)
        e_fwd = e_fwd_new

        parts = _sc_scatter(n, msg, idx_dst_s, msg, idx_src_s, zeros, e)
        a0, a1 = parts[0], parts[1]

        (v1, c1), (v2, c2) = layer["node_mlp"]["layers"]
        vg, vbeta = layer["node_mlp"]["ln"]
        v1a, v1b = v1[:H], v1[H:]
        h = _node_call(h, a0, a1, v1a, v1b, v2,
                       _aux_pack(c1, c2, vg, vbeta))

    fparts = _sc_scatter(n, e_fwd, idx_dst_s, e_bwd_f, idx_src_s, zeros, e)
    i0, i1 = fparts[0], fparts[1]

    fg, fbeta = params["final_ln"]
    lnaux = jnp.stack([fg, fbeta])
    lnaux = jnp.concatenate([lnaux, jnp.zeros((6, 2 * H), jnp.float32)])

    (aux_w1, aux_b1), (aux_w2, aux_b2) = params["dec_ux"]["layers"]
    (auz_w1, auz_b1), (auz_w2, auz_b2) = params["dec_uz"]["layers"]
    (ath_w1, ath_b1), (ath_w2, ath_b2) = params["dec_th"]["layers"]
    acomb = jnp.concatenate([aux_w1, auz_w1, ath_w1], axis=1)
    abias = jnp.concatenate([aux_b1, auz_b1, ath_b1])
    aaux = _aux_pack(abias)
    bblk = jnp.zeros((3 * H, H), jnp.float32)
    bblk = bblk.at[0:H, 0:1].set(aux_w2)
    bblk = bblk.at[H:2 * H, 1:2].set(auz_w2)
    bblk = bblk.at[2 * H:3 * H, 2:3].set(ath_w2)
    bbias = jnp.zeros((H,), jnp.float32)
    bbias = bbias.at[0].set(aux_b2[0]).at[1].set(auz_b2[0]).at[2].set(ath_b2[0])
    baux = _aux_pack(bbias)

    pred = _final_call(h, i0, i1, bc_disp, bc_rot,
                       lnaux, acomb, aaux, bblk, baux)
    return pred[:, :3]


# speedup vs baseline: 3.7134x; 1.0728x over previous
import functools

import jax
import jax.numpy as jnp
from jax import lax
from jax.experimental import pallas as pl
from jax.experimental.pallas import tpu as pltpu
from jax.experimental.pallas import tpu_sc as plsc

H = 128
EBLK = 2000
NBLK = 2000


def _celu(x):
    return jnp.where(x > 0, x, jnp.exp(jnp.minimum(x, 0.0)) - 1.0)


def _ln(x, g, beta):
    mu = jnp.mean(x, axis=-1, keepdims=True)
    xc = x - mu
    var = jnp.mean(xc * xc, axis=-1, keepdims=True)
    return xc * jax.lax.rsqrt(var + 1e-5) * g + beta


def _aux_pack(*rows):
    d = rows[0].shape[0]
    a = jnp.zeros((8, d), jnp.float32)
    for i, r in enumerate(rows):
        a = a.at[i].set(r)
    return a


def _enc_body(x_ref, w1_ref, w2_ref, aux_ref, out_ref, *, kdim):
    x = x_ref[...]
    acc = jnp.zeros((x.shape[0], H), jnp.float32) + aux_ref[0:1, :]
    for k in range(kdim):
        acc = acc + x[:, k:k + 1] * w1_ref[k:k + 1, :]
    t = _celu(acc)
    y = jnp.dot(t, w2_ref[...], preferred_element_type=jnp.float32)
    y = y + aux_ref[1:2, :]
    out_ref[...] = _ln(y, aux_ref[2:3, :], aux_ref[3:4, :])


def _pq_body(h_ref, w1b_ref, w1c_ref, p_ref, q_ref):
    h = h_ref[...]
    p_ref[...] = jnp.dot(h, w1b_ref[...], preferred_element_type=jnp.float32)
    q_ref[...] = jnp.dot(h, w1c_ref[...], preferred_element_type=jnp.float32)


def _edge_body(ea_ref, r_ref, w1a_ref, w2_ref, aux_ref,
               msg_ref, enew_ref):
    ea = ea_ref[...]
    t = jnp.dot(ea, w1a_ref[...], preferred_element_type=jnp.float32)
    t = t + r_ref[...] + aux_ref[0:1, :]
    t = _celu(t)
    y = jnp.dot(t, w2_ref[...], preferred_element_type=jnp.float32)
    y = y + aux_ref[1:2, :]
    m = _ln(y, aux_ref[2:3, :], aux_ref[3:4, :])
    msg_ref[...] = m
    enew_ref[...] = ea + m


def _edge_last_body(ea_ref, r_ref, ef0_ref, eb0_ref,
                    w1a_ref, w2_ref, aux_ref,
                    msg_ref, enew_ref, ebwd_ref):
    ea = ea_ref[...]
    t = jnp.dot(ea, w1a_ref[...], preferred_element_type=jnp.float32)
    t = t + r_ref[...] + aux_ref[0:1, :]
    t = _celu(t)
    y = jnp.dot(t, w2_ref[...], preferred_element_type=jnp.float32)
    y = y + aux_ref[1:2, :]
    m = _ln(y, aux_ref[2:3, :], aux_ref[3:4, :])
    enew = ea + m
    msg_ref[...] = m
    enew_ref[...] = enew
    ebwd_ref[...] = eb0_ref[...] + ef0_ref[...] - enew


def _node_body(h_ref, a0_ref, a1_ref, v1a_ref, v1b_ref, v2_ref, aux_ref,
               out_ref):
    h = h_ref[...]
    agg = a0_ref[...] - a1_ref[...]
    t = jnp.dot(h, v1a_ref[...], preferred_element_type=jnp.float32)
    t = t + jnp.dot(agg, v1b_ref[...], preferred_element_type=jnp.float32)
    t = _celu(t + aux_ref[0:1, :])
    y = jnp.dot(t, v2_ref[...], preferred_element_type=jnp.float32)
    y = y + aux_ref[1:2, :]
    out_ref[...] = h + _ln(y, aux_ref[2:3, :], aux_ref[3:4, :])


def _final_body(h_ref, i0_ref, i1_ref, d_ref, r_ref,
                lnaux_ref, acomb_ref, aaux_ref, bblk_ref, baux_ref,
                out_ref):
    inc = i0_ref[...] + i1_ref[...]
    z = jnp.concatenate([h_ref[...], inc], axis=1)
    z = _ln(z, lnaux_ref[0:1, :], lnaux_ref[1:2, :])
    t = jnp.dot(z, acomb_ref[...], preferred_element_type=jnp.float32)
    t = _celu(t + aaux_ref[0:1, :])
    y = jnp.dot(t, bblk_ref[...], preferred_element_type=jnp.float32)
    y = y + baux_ref[0:1, :]
    dm = 1.0 - d_ref[...]
    rm = 1.0 - r_ref[...]
    col = jax.lax.broadcasted_iota(jnp.int32, y.shape, 1)
    mask = jnp.where(col < 2, dm, jnp.where(col < 3, rm, 0.0))
    out_ref[...] = y * mask


def _full(shape):
    return pl.BlockSpec(shape, lambda i: (0, 0))


def _blk(m, pref):
    return pref if m % pref == 0 else m


def _rows(blk, width):
    return pl.BlockSpec((blk, width), lambda i: (i, 0))


def _enc_call(xp, w1, w2, aux, blk):
    m = xp.shape[0]
    kdim = w1.shape[0]
    return pl.pallas_call(
        functools.partial(_enc_body, kdim=kdim),
        grid=(m // blk,),
        in_specs=[_rows(blk, xp.shape[1]), _full(w1.shape), _full(w2.shape),
                  _full(aux.shape)],
        out_specs=_rows(blk, H),
        out_shape=jax.ShapeDtypeStruct((m, H), jnp.float32),
    )(xp, w1, w2, aux)


def _pq_call(h, w1b, w1c):
    n = h.shape[0]
    blk = _blk(n, NBLK)
    return pl.pallas_call(
        _pq_body,
        grid=(n // blk,),
        in_specs=[_rows(blk, H), _full((H, H)), _full((H, H))],
        out_specs=[_rows(blk, H), _rows(blk, H)],
        out_shape=[jax.ShapeDtypeStruct((n, H), jnp.float32),
                   jax.ShapeDtypeStruct((n, H), jnp.float32)],
    )(h, w1b, w1c)


def _edge_call(ea, r, w1a, w2, aux):
    e = ea.shape[0]
    blk = _blk(e, EBLK)
    return pl.pallas_call(
        _edge_body,
        grid=(e // blk,),
        in_specs=[_rows(blk, H)] * 2 + [_full((H, H)), _full((H, H)),
                                        _full(aux.shape)],
        out_specs=[_rows(blk, H), _rows(blk, H)],
        out_shape=[jax.ShapeDtypeStruct((e, H), jnp.float32),
                   jax.ShapeDtypeStruct((e, H), jnp.float32)],
    )(ea, r, w1a, w2, aux)


def _edge_last_call(ea, r, ef0, eb0, w1a, w2, aux):
    e = ea.shape[0]
    blk = _blk(e, EBLK)
    return pl.pallas_call(
        _edge_last_body,
        grid=(e // blk,),
        in_specs=[_rows(blk, H)] * 4 + [_full((H, H)), _full((H, H)),
                                        _full(aux.shape)],
        out_specs=[_rows(blk, H)] * 3,
        out_shape=[jax.ShapeDtypeStruct((e, H), jnp.float32)] * 3,
    )(ea, r, ef0, eb0, w1a, w2, aux)


def _node_call(h, a0, a1, v1a, v1b, v2, aux):
    n = h.shape[0]
    blk = _blk(n, NBLK)
    return pl.pallas_call(
        _node_body,
        grid=(n // blk,),
        in_specs=[_rows(blk, H)] * 3 + [_full((H, H))] * 3 + [_full(aux.shape)],
        out_specs=_rows(blk, H),
        out_shape=jax.ShapeDtypeStruct((n, H), jnp.float32),
    )(h, a0, a1, v1a, v1b, v2, aux)


def _final_call(h, i0, i1, d, r, lnaux, acomb, aaux, bblk, baux):
    n = h.shape[0]
    blk = _blk(n, NBLK)
    return pl.pallas_call(
        _final_body,
        grid=(n // blk,),
        in_specs=[_rows(blk, H)] * 3 + [_rows(blk, 1)] * 2 +
                 [_full(lnaux.shape), _full(acomb.shape), _full(aaux.shape),
                  _full(bblk.shape), _full(baux.shape)],
        out_specs=_rows(blk, H),
        out_shape=jax.ShapeDtypeStruct((n, H), jnp.float32),
    )(h, i0, i1, d, r, lnaux, acomb, aaux, bblk, baux)


_NC = 2
_NS = 16
_CHUNK = 128


_GG = 6
_GS = 3


def _sc_gather(p, q, idxd2, idxs2, e):
    nchunk = e // _CHUNK
    nsc = -(-nchunk // _GG)
    nw = _NC * _NS
    per_w = -(-nsc // nw)
    mesh = plsc.VectorSubcoreMesh(core_axis_name="c", subcore_axis_name="s")

    @functools.partial(
        pl.kernel, mesh=mesh,
        out_type=jax.ShapeDtypeStruct((e, H), jnp.float32),
        scratch_types=[pltpu.VMEM((_GG, _CHUNK), jnp.int32),
                       pltpu.VMEM((_GG, _CHUNK), jnp.int32),
                       pltpu.VMEM((_GG * _CHUNK, H), jnp.float32),
                       pltpu.SemaphoreType.DMA,
                       pltpu.SemaphoreType.DMA,
                       pltpu.SemaphoreType.DMA],
    )
    def k(p_hbm, q_hbm, idxd_hbm, idxs_hbm, r_hbm,
          idxd_v, idxs_v, rows_v, sem1, sem2, sem3):
        wid = lax.axis_index("s") * _NC + lax.axis_index("c")

        def body(t, carry):
            sc = t * nw + wid

            @pl.when(sc < nsc)
            def _():
                pltpu.sync_copy(idxd_hbm.at[sc], idxd_v)
                pltpu.sync_copy(idxs_hbm.at[sc], idxs_v)
                for j in range(_GG):
                    @pl.when(sc * _GG + j < nchunk)
                    def _(j=j):
                        pltpu.async_copy(
                            p_hbm.at[idxd_v.at[j]],
                            rows_v.at[pl.ds(j * _CHUNK, _CHUNK)], sem1)
                for j in range(_GG):
                    @pl.when(sc * _GG + j < nchunk)
                    def _(j=j):
                        pltpu.make_async_copy(
                            p_hbm.at[idxd_v.at[j]],
                            rows_v.at[pl.ds(j * _CHUNK, _CHUNK)], sem1).wait()
                for j in range(_GG):
                    @pl.when(sc * _GG + j < nchunk)
                    def _(j=j):
                        pltpu.async_copy(
                            q_hbm.at[idxs_v.at[j]],
                            rows_v.at[pl.ds(j * _CHUNK, _CHUNK)], sem2,
                            add=True)
                for j in range(_GG):
                    @pl.when(sc * _GG + j < nchunk)
                    def _(j=j):
                        pltpu.make_async_copy(
                            q_hbm.at[idxs_v.at[j]],
                            rows_v.at[pl.ds(j * _CHUNK, _CHUNK)], sem2).wait()
                for j in range(_GG):
                    @pl.when(sc * _GG + j < nchunk)
                    def _(j=j):
                        pltpu.async_copy(
                            rows_v.at[pl.ds(j * _CHUNK, _CHUNK)],
                            r_hbm.at[pl.ds((sc * _GG + j) * _CHUNK, _CHUNK)],
                            sem3)
                for j in range(_GG):
                    @pl.when(sc * _GG + j < nchunk)
                    def _(j=j):
                        pltpu.make_async_copy(
                            rows_v.at[pl.ds(j * _CHUNK, _CHUNK)],
                            r_hbm.at[pl.ds((sc * _GG + j) * _CHUNK, _CHUNK)],
                            sem3).wait()

            return carry

        lax.fori_loop(0, per_w, body, 0)

    return k(p, q, idxd2, idxs2)


def _sc_scatter(n, rows0, idx0_2, rows1, idx1_2, zeros, e):
    nchunk = e // _CHUNK
    nsc = -(-nchunk // _GS)
    per_t = -(-nsc // _NS)
    rpt = -(-n // (8 * _NS)) * 8
    acc_n = rpt * _NS
    tail = n - rpt * (_NS - 1)
    mesh = plsc.VectorSubcoreMesh(core_axis_name="c", subcore_axis_name="s")

    @functools.partial(
        pl.kernel, mesh=mesh,
        out_type=[jax.ShapeDtypeStruct((n, H), jnp.float32),
                  jax.ShapeDtypeStruct((n, H), jnp.float32)],
        scratch_types=[pltpu.VMEM((_GS, _CHUNK), jnp.int32),
                       pltpu.VMEM((_GS * _CHUNK, H), jnp.float32),
                       pltpu.VMEM_SHARED((acc_n, H), jnp.float32),
                       pltpu.SemaphoreType.DMA,
                       pltpu.SemaphoreType.DMA],
    )
    def k(r0_hbm, r1_hbm, i0_hbm, i1_hbm, z_hbm, out0_hbm, out1_hbm,
          idx_v, rows_v, acc_sh, semA, semB):
        c = lax.axis_index("c")
        s = lax.axis_index("s")
        rbase = s * rpt

        @pl.when(s < _NS - 1)
        def _():
            pltpu.sync_copy(z_hbm.at[pl.ds(rbase, rpt)],
                            acc_sh.at[pl.ds(rbase, rpt)])

        @pl.when(s == _NS - 1)
        def _():
            pltpu.sync_copy(z_hbm.at[pl.ds((_NS - 1) * rpt, tail)],
                            acc_sh.at[pl.ds((_NS - 1) * rpt, tail)])

        plsc.subcore_barrier()

        def body(t, carry):
            sc = t * _NS + s

            @pl.when(sc < nsc)
            def _():
                @pl.when(c == 0)
                def _():
                    pltpu.sync_copy(i0_hbm.at[sc], idx_v)
                    for j in range(_GS):
                        @pl.when(sc * _GS + j < nchunk)
                        def _(j=j):
                            pltpu.async_copy(
                                r0_hbm.at[pl.ds((sc * _GS + j) * _CHUNK, _CHUNK)],
                                rows_v.at[pl.ds(j * _CHUNK, _CHUNK)], semA)

                @pl.when(c == 1)
                def _():
                    pltpu.sync_copy(i1_hbm.at[sc], idx_v)
                    for j in range(_GS):
                        @pl.when(sc * _GS + j < nchunk)
                        def _(j=j):
                            pltpu.async_copy(
                                r1_hbm.at[pl.ds((sc * _GS + j) * _CHUNK, _CHUNK)],
                                rows_v.at[pl.ds(j * _CHUNK, _CHUNK)], semA)

                for j in range(_GS):
                    @pl.when(sc * _GS + j < nchunk)
                    def _(j=j):
                        pltpu.make_async_copy(
                            r0_hbm.at[pl.ds((sc * _GS + j) * _CHUNK, _CHUNK)],
                            rows_v.at[pl.ds(j * _CHUNK, _CHUNK)], semA).wait()
                for j in range(_GS):
                    @pl.when(sc * _GS + j < nchunk)
                    def _(j=j):
                        pltpu.async_copy(
                            rows_v.at[pl.ds(j * _CHUNK, _CHUNK)],
                            acc_sh.at[idx_v.at[j]], semB, add=True)
                for j in range(_GS):
                    @pl.when(sc * _GS + j < nchunk)
                    def _(j=j):
                        pltpu.make_async_copy(
                            rows_v.at[pl.ds(j * _CHUNK, _CHUNK)],
                            acc_sh.at[idx_v.at[j]], semB).wait()

            return carry

        lax.fori_loop(0, per_t, body, 0)
        plsc.subcore_barrier()

        @pl.when(jnp.logical_and(s < _NS - 1, c == 0))
        def _():
            pltpu.sync_copy(acc_sh.at[pl.ds(rbase, rpt)],
                            out0_hbm.at[pl.ds(rbase, rpt)])

        @pl.when(jnp.logical_and(s == _NS - 1, c == 0))
        def _():
            pltpu.sync_copy(acc_sh.at[pl.ds((_NS - 1) * rpt, tail)],
                            out0_hbm.at[pl.ds((_NS - 1) * rpt, tail)])

        @pl.when(jnp.logical_and(s < _NS - 1, c == 1))
        def _():
            pltpu.sync_copy(acc_sh.at[pl.ds(rbase, rpt)],
                            out1_hbm.at[pl.ds(rbase, rpt)])

        @pl.when(jnp.logical_and(s == _NS - 1, c == 1))
        def _():
            pltpu.sync_copy(acc_sh.at[pl.ds((_NS - 1) * rpt, tail)],
                            out1_hbm.at[pl.ds((_NS - 1) * rpt, tail)])

    return k(rows0, rows1, idx0_2, idx1_2, zeros)


def kernel(x, edge_index, edge_attr, n_elements, bc_disp, bc_rot, params):
    n = x.shape[0]
    e2 = edge_attr.shape[0]
    e = e2 // 2

    idx_src = edge_index[0, :e].astype(jnp.int32)
    idx_dst = edge_index[1, :e].astype(jnp.int32)
    zeros = jnp.zeros((n, H), jnp.float32)
    nchunk = e // _CHUNK

    def _pad3(idx, g):
        nsc = -(-nchunk // g)
        return jnp.pad(idx.reshape(nchunk, _CHUNK),
                       ((0, nsc * g - nchunk), (0, 0))).reshape(nsc, g, _CHUNK)

    idx_dst_g = _pad3(idx_dst, _GG)
    idx_src_g = _pad3(idx_src, _GG)
    idx_dst_s = _pad3(idx_dst, _GS)
    idx_src_s = _pad3(idx_src, _GS)

    ne = params["node_enc"]
    (nw1, nb1), (nw2, nb2) = ne["layers"]
    ng, nbeta = ne["ln"]
    xp = jnp.pad(x, ((0, 0), (0, 16 - x.shape[1])))
    nw1p = jnp.pad(nw1, ((0, 16 - nw1.shape[0]), (0, 0)))
    h = _enc_call(xp, nw1p, nw2, _aux_pack(nb1, nb2, ng, nbeta),
                  _blk(x.shape[0], NBLK))

    ee = params["edge_enc"]
    (ew1, eb1), (ew2, eb2) = ee["layers"]
    eg, ebeta = ee["ln"]
    eap = jnp.pad(edge_attr, ((0, 0), (0, 8 - edge_attr.shape[1])))
    ew1p = jnp.pad(ew1, ((0, 8 - ew1.shape[0]), (0, 0)))
    eaux = _aux_pack(eb1, eb2, eg, ebeta)
    e_fwd0 = _enc_call(eap[:e], ew1p, ew2, eaux, _blk(e, EBLK))
    e_bwd0 = _enc_call(eap[e:], ew1p, ew2, eaux, _blk(e, EBLK))

    e_fwd = e_fwd0
    n_layers = len(params["mp"])
    for li, layer in enumerate(params["mp"]):
        (w1, b1), (w2, b2) = layer["edge_mlp"]["layers"]
        g, beta = layer["edge_mlp"]["ln"]
        w1a, w1b, w1c = w1[:H], w1[H:2 * H], w1[2 * H:]
        eaux = _aux_pack(b1, b2, g, beta)

        p, q = _pq_call(h, w1b, w1c)
        r = _sc_gather(p, q, idx_dst_g, idx_src_g, e)

        if li == n_layers - 1:
            msg, e_fwd_new, e_bwd_f = _edge_last_call(
                e_fwd, r, e_fwd0, e_bwd0, w1a, w2, eaux)
        else:
            msg, e_fwd_new = _edge_call(e_fwd, r, w1a, w2, eaux)
        e_fwd = e_fwd_new

        a0, a1 = _sc_scatter(n, msg, idx_dst_s, msg, idx_src_s, zeros, e)

        (v1, c1), (v2, c2) = layer["node_mlp"]["layers"]
        vg, vbeta = layer["node_mlp"]["ln"]
        v1a, v1b = v1[:H], v1[H:]
        h = _node_call(h, a0, a1, v1a, v1b, v2,
                       _aux_pack(c1, c2, vg, vbeta))

    i0, i1 = _sc_scatter(n, e_fwd, idx_dst_s, e_bwd_f, idx_src_s, zeros, e)

    fg, fbeta = params["final_ln"]
    lnaux = jnp.stack([fg, fbeta])
    lnaux = jnp.concatenate([lnaux, jnp.zeros((6, 2 * H), jnp.float32)])

    (aux_w1, aux_b1), (aux_w2, aux_b2) = params["dec_ux"]["layers"]
    (auz_w1, auz_b1), (auz_w2, auz_b2) = params["dec_uz"]["layers"]
    (ath_w1, ath_b1), (ath_w2, ath_b2) = params["dec_th"]["layers"]
    acomb = jnp.concatenate([aux_w1, auz_w1, ath_w1], axis=1)
    abias = jnp.concatenate([aux_b1, auz_b1, ath_b1])
    aaux = _aux_pack(abias)
    bblk = jnp.zeros((3 * H, H), jnp.float32)
    bblk = bblk.at[0:H, 0:1].set(aux_w2)
    bblk = bblk.at[H:2 * H, 1:2].set(auz_w2)
    bblk = bblk.at[2 * H:3 * H, 2:3].set(ath_w2)
    bbias = jnp.zeros((H,), jnp.float32)
    bbias = bbias.at[0].set(aux_b2[0]).at[1].set(auz_b2[0]).at[2].set(ath_b2[0])
    baux = _aux_pack(bbias)

    pred = _final_call(h, i0, i1, bc_disp, bc_rot,
                       lnaux, acomb, aaux, bblk, baux)
    return pred[:, :3]


# speedup vs baseline: 3.7203x; 1.0019x over previous
import functools

import jax
import jax.numpy as jnp
from jax import lax
from jax.experimental import pallas as pl
from jax.experimental.pallas import tpu as pltpu
from jax.experimental.pallas import tpu_sc as plsc

H = 128
EBLK = 2000
NBLK = 2000


def _celu(x):
    return jnp.where(x > 0, x, jnp.exp(jnp.minimum(x, 0.0)) - 1.0)


def _ln(x, g, beta):
    mu = jnp.mean(x, axis=-1, keepdims=True)
    xc = x - mu
    var = jnp.mean(xc * xc, axis=-1, keepdims=True)
    return xc * jax.lax.rsqrt(var + 1e-5) * g + beta


def _aux_pack(*rows):
    d = rows[0].shape[0]
    a = jnp.zeros((8, d), jnp.float32)
    for i, r in enumerate(rows):
        a = a.at[i].set(r)
    return a


def _enc_body(x_ref, w1_ref, w2_ref, aux_ref, out_ref, *, kdim):
    x = x_ref[...]
    acc = jnp.zeros((x.shape[0], H), jnp.float32) + aux_ref[0:1, :]
    for k in range(kdim):
        acc = acc + x[:, k:k + 1] * w1_ref[k:k + 1, :]
    t = _celu(acc)
    y = jnp.dot(t, w2_ref[...], preferred_element_type=jnp.float32)
    y = y + aux_ref[1:2, :]
    out_ref[...] = _ln(y, aux_ref[2:3, :], aux_ref[3:4, :])


def _pq_body(h_ref, w1b_ref, w1c_ref, p_ref, q_ref):
    h = h_ref[...]
    p_ref[...] = jnp.dot(h, w1b_ref[...], preferred_element_type=jnp.float32)
    q_ref[...] = jnp.dot(h, w1c_ref[...], preferred_element_type=jnp.float32)


def _edge_body(ea_ref, r_ref, w1a_ref, w2_ref, aux_ref,
               msg_ref, enew_ref):
    ea = ea_ref[...]
    t = jnp.dot(ea, w1a_ref[...], preferred_element_type=jnp.float32)
    t = t + r_ref[...] + aux_ref[0:1, :]
    t = _celu(t)
    y = jnp.dot(t, w2_ref[...], preferred_element_type=jnp.float32)
    y = y + aux_ref[1:2, :]
    m = _ln(y, aux_ref[2:3, :], aux_ref[3:4, :])
    msg_ref[...] = m
    enew_ref[...] = ea + m


def _edge_last_body(ea_ref, r_ref, ef0_ref, eb0_ref,
                    w1a_ref, w2_ref, aux_ref,
                    msg_ref, enew_ref, ebwd_ref):
    ea = ea_ref[...]
    t = jnp.dot(ea, w1a_ref[...], preferred_element_type=jnp.float32)
    t = t + r_ref[...] + aux_ref[0:1, :]
    t = _celu(t)
    y = jnp.dot(t, w2_ref[...], preferred_element_type=jnp.float32)
    y = y + aux_ref[1:2, :]
    m = _ln(y, aux_ref[2:3, :], aux_ref[3:4, :])
    enew = ea + m
    msg_ref[...] = m
    enew_ref[...] = enew
    ebwd_ref[...] = eb0_ref[...] + ef0_ref[...] - enew


def _node_body(h_ref, a0_ref, a1_ref, v1a_ref, v1b_ref, v2_ref, aux_ref,
               out_ref):
    h = h_ref[...]
    agg = a0_ref[...] - a1_ref[...]
    t = jnp.dot(h, v1a_ref[...], preferred_element_type=jnp.float32)
    t = t + jnp.dot(agg, v1b_ref[...], preferred_element_type=jnp.float32)
    t = _celu(t + aux_ref[0:1, :])
    y = jnp.dot(t, v2_ref[...], preferred_element_type=jnp.float32)
    y = y + aux_ref[1:2, :]
    out_ref[...] = h + _ln(y, aux_ref[2:3, :], aux_ref[3:4, :])


def _final_body(h_ref, i0_ref, i1_ref, d_ref, r_ref,
                lnaux_ref, acomb_ref, aaux_ref, bblk_ref, baux_ref,
                out_ref):
    inc = i0_ref[...] + i1_ref[...]
    z = jnp.concatenate([h_ref[...], inc], axis=1)
    z = _ln(z, lnaux_ref[0:1, :], lnaux_ref[1:2, :])
    t = jnp.dot(z, acomb_ref[...], preferred_element_type=jnp.float32)
    t = _celu(t + aaux_ref[0:1, :])
    y = jnp.dot(t, bblk_ref[...], preferred_element_type=jnp.float32)
    y = y + baux_ref[0:1, :]
    dm = 1.0 - d_ref[...]
    rm = 1.0 - r_ref[...]
    col = jax.lax.broadcasted_iota(jnp.int32, y.shape, 1)
    mask = jnp.where(col < 2, dm, jnp.where(col < 3, rm, 0.0))
    out_ref[...] = y * mask


def _full(shape):
    return pl.BlockSpec(shape, lambda i: (0, 0))


def _blk(m, pref):
    return pref if m % pref == 0 else m


def _rows(blk, width):
    return pl.BlockSpec((blk, width), lambda i: (i, 0))


def _enc_call(xp, w1, w2, aux, blk):
    m = xp.shape[0]
    kdim = w1.shape[0]
    return pl.pallas_call(
        functools.partial(_enc_body, kdim=kdim),
        grid=(m // blk,),
        in_specs=[_rows(blk, xp.shape[1]), _full(w1.shape), _full(w2.shape),
                  _full(aux.shape)],
        out_specs=_rows(blk, H),
        out_shape=jax.ShapeDtypeStruct((m, H), jnp.float32),
    )(xp, w1, w2, aux)


def _pq_call(h, w1b, w1c):
    n = h.shape[0]
    blk = _blk(n, NBLK)
    return pl.pallas_call(
        _pq_body,
        grid=(n // blk,),
        in_specs=[_rows(blk, H), _full((H, H)), _full((H, H))],
        out_specs=[_rows(blk, H), _rows(blk, H)],
        out_shape=[jax.ShapeDtypeStruct((n, H), jnp.float32),
                   jax.ShapeDtypeStruct((n, H), jnp.float32)],
    )(h, w1b, w1c)


def _edge_call(ea, r, w1a, w2, aux):
    e = ea.shape[0]
    blk = _blk(e, EBLK)
    return pl.pallas_call(
        _edge_body,
        grid=(e // blk,),
        in_specs=[_rows(blk, H)] * 2 + [_full((H, H)), _full((H, H)),
                                        _full(aux.shape)],
        out_specs=[_rows(blk, H), _rows(blk, H)],
        out_shape=[jax.ShapeDtypeStruct((e, H), jnp.float32),
                   jax.ShapeDtypeStruct((e, H), jnp.float32)],
    )(ea, r, w1a, w2, aux)


def _edge_last_call(ea, r, ef0, eb0, w1a, w2, aux):
    e = ea.shape[0]
    blk = _blk(e, EBLK)
    return pl.pallas_call(
        _edge_last_body,
        grid=(e // blk,),
        in_specs=[_rows(blk, H)] * 4 + [_full((H, H)), _full((H, H)),
                                        _full(aux.shape)],
        out_specs=[_rows(blk, H)] * 3,
        out_shape=[jax.ShapeDtypeStruct((e, H), jnp.float32)] * 3,
    )(ea, r, ef0, eb0, w1a, w2, aux)


def _node_call(h, a0, a1, v1a, v1b, v2, aux):
    n = h.shape[0]
    blk = _blk(n, NBLK)
    return pl.pallas_call(
        _node_body,
        grid=(n // blk,),
        in_specs=[_rows(blk, H)] * 3 + [_full((H, H))] * 3 + [_full(aux.shape)],
        out_specs=_rows(blk, H),
        out_shape=jax.ShapeDtypeStruct((n, H), jnp.float32),
    )(h, a0, a1, v1a, v1b, v2, aux)


def _final_call(h, i0, i1, d, r, lnaux, acomb, aaux, bblk, baux):
    n = h.shape[0]
    blk = _blk(n, NBLK)
    return pl.pallas_call(
        _final_body,
        grid=(n // blk,),
        in_specs=[_rows(blk, H)] * 3 + [_rows(blk, 1)] * 2 +
                 [_full(lnaux.shape), _full(acomb.shape), _full(aaux.shape),
                  _full(bblk.shape), _full(baux.shape)],
        out_specs=_rows(blk, H),
        out_shape=jax.ShapeDtypeStruct((n, H), jnp.float32),
    )(h, i0, i1, d, r, lnaux, acomb, aaux, bblk, baux)


_NC = 2
_NS = 16
_CHUNK = 128


_GP = 3
_GS = 3


def _sc_gather(p, q, idxd2, idxs2, e):
    nchunk = e // _CHUNK
    nsc = -(-nchunk // _GP)
    nw = _NC * _NS
    per_w = -(-nsc // nw)
    niter = -(-per_w // 2)
    mesh = plsc.VectorSubcoreMesh(core_axis_name="c", subcore_axis_name="s")

    @functools.partial(
        pl.kernel, mesh=mesh,
        out_type=jax.ShapeDtypeStruct((e, H), jnp.float32),
        scratch_types=[pltpu.VMEM((_GP, _CHUNK), jnp.int32),
                       pltpu.VMEM((_GP, _CHUNK), jnp.int32),
                       pltpu.VMEM((_GP, _CHUNK), jnp.int32),
                       pltpu.VMEM((_GP, _CHUNK), jnp.int32),
                       pltpu.VMEM((_GP * _CHUNK, H), jnp.float32),
                       pltpu.VMEM((_GP * _CHUNK, H), jnp.float32),
                       pltpu.SemaphoreType.DMA,
                       pltpu.SemaphoreType.DMA,
                       pltpu.SemaphoreType.DMA,
                       pltpu.SemaphoreType.DMA],
    )
    def k(p_hbm, q_hbm, idxd_hbm, idxs_hbm, r_hbm,
          idxd0_v, idxs0_v, idxd1_v, idxs1_v, rows0_v, rows1_v,
          sem1, sem2, sem3a, sem3b):
        wid = lax.axis_index("s") * _NC + lax.axis_index("c")
        bufs = ((idxd0_v, idxs0_v, rows0_v, sem3a),
                (idxd1_v, idxs1_v, rows1_v, sem3b))

        def body(it, carry):
            for u in range(2):
                idxd_v, idxs_v, rows_v, sem3 = bufs[u]
                t = it * 2 + u
                sc = t * nw + wid
                sc_prev = (t - 2) * nw + wid

                for j in range(_GP):
                    @pl.when(jnp.logical_and(
                        t >= 2, sc_prev * _GP + j < nchunk))
                    def _(j=j):
                        pltpu.make_async_copy(
                            rows_v.at[pl.ds(j * _CHUNK, _CHUNK)],
                            r_hbm.at[pl.ds((sc_prev * _GP + j) * _CHUNK,
                                           _CHUNK)], sem3).wait()

                @pl.when(sc < nsc)
                def _():
                    pltpu.sync_copy(idxd_hbm.at[sc], idxd_v)
                    pltpu.sync_copy(idxs_hbm.at[sc], idxs_v)
                    for j in range(_GP):
                        @pl.when(sc * _GP + j < nchunk)
                        def _(j=j):
                            pltpu.async_copy(
                                p_hbm.at[idxd_v.at[j]],
                                rows_v.at[pl.ds(j * _CHUNK, _CHUNK)], sem1)
                    for j in range(_GP):
                        @pl.when(sc * _GP + j < nchunk)
                        def _(j=j):
                            pltpu.make_async_copy(
                                p_hbm.at[idxd_v.at[j]],
                                rows_v.at[pl.ds(j * _CHUNK, _CHUNK)],
                                sem1).wait()
                    for j in range(_GP):
                        @pl.when(sc * _GP + j < nchunk)
                        def _(j=j):
                            pltpu.async_copy(
                                q_hbm.at[idxs_v.at[j]],
                                rows_v.at[pl.ds(j * _CHUNK, _CHUNK)], sem2,
                                add=True)
                    for j in range(_GP):
                        @pl.when(sc * _GP + j < nchunk)
                        def _(j=j):
                            pltpu.make_async_copy(
                                q_hbm.at[idxs_v.at[j]],
                                rows_v.at[pl.ds(j * _CHUNK, _CHUNK)],
                                sem2).wait()
                    for j in range(_GP):
                        @pl.when(sc * _GP + j < nchunk)
                        def _(j=j):
                            pltpu.async_copy(
                                rows_v.at[pl.ds(j * _CHUNK, _CHUNK)],
                                r_hbm.at[pl.ds((sc * _GP + j) * _CHUNK,
                                               _CHUNK)], sem3)

            return carry

        lax.fori_loop(0, niter, body, 0)

        for u in range(2):
            idxd_v, idxs_v, rows_v, sem3 = bufs[u]
            t_last = (niter - 1) * 2 + u
            sc_last = t_last * nw + wid
            for j in range(_GP):
                @pl.when(sc_last * _GP + j < nchunk)
                def _(j=j):
                    pltpu.make_async_copy(
                        rows_v.at[pl.ds(j * _CHUNK, _CHUNK)],
                        r_hbm.at[pl.ds((sc_last * _GP + j) * _CHUNK,
                                       _CHUNK)], sem3).wait()
    return k(p, q, idxd2, idxs2)


def _sc_scatter(n, rows0, idx0_2, rows1, idx1_2, zeros, e):
    nchunk = e // _CHUNK
    nsc = -(-nchunk // _GS)
    per_t = -(-nsc // _NS)
    rpt = -(-n // (8 * _NS)) * 8
    acc_n = rpt * _NS
    tail = n - rpt * (_NS - 1)
    mesh = plsc.VectorSubcoreMesh(core_axis_name="c", subcore_axis_name="s")

    @functools.partial(
        pl.kernel, mesh=mesh,
        out_type=[jax.ShapeDtypeStruct((n, H), jnp.float32),
                  jax.ShapeDtypeStruct((n, H), jnp.float32)],
        scratch_types=[pltpu.VMEM((_GS, _CHUNK), jnp.int32),
                       pltpu.VMEM((_GS * _CHUNK, H), jnp.float32),
                       pltpu.VMEM_SHARED((acc_n, H), jnp.float32),
                       pltpu.SemaphoreType.DMA,
                       pltpu.SemaphoreType.DMA],
    )
    def k(r0_hbm, r1_hbm, i0_hbm, i1_hbm, z_hbm, out0_hbm, out1_hbm,
          idx_v, rows_v, acc_sh, semA, semB):
        c = lax.axis_index("c")
        s = lax.axis_index("s")
        rbase = s * rpt

        @pl.when(s < _NS - 1)
        def _():
            pltpu.sync_copy(z_hbm.at[pl.ds(rbase, rpt)],
                            acc_sh.at[pl.ds(rbase, rpt)])

        @pl.when(s == _NS - 1)
        def _():
            pltpu.sync_copy(z_hbm.at[pl.ds((_NS - 1) * rpt, tail)],
                            acc_sh.at[pl.ds((_NS - 1) * rpt, tail)])

        plsc.subcore_barrier()

        def body(t, carry):
            sc = t * _NS + s

            @pl.when(sc < nsc)
            def _():
                @pl.when(c == 0)
                def _():
                    pltpu.sync_copy(i0_hbm.at[sc], idx_v)
                    for j in range(_GS):
                        @pl.when(sc * _GS + j < nchunk)
                        def _(j=j):
                            pltpu.async_copy(
                                r0_hbm.at[pl.ds((sc * _GS + j) * _CHUNK, _CHUNK)],
                                rows_v.at[pl.ds(j * _CHUNK, _CHUNK)], semA)

                @pl.when(c == 1)
                def _():
                    pltpu.sync_copy(i1_hbm.at[sc], idx_v)
                    for j in range(_GS):
                        @pl.when(sc * _GS + j < nchunk)
                        def _(j=j):
                            pltpu.async_copy(
                                r1_hbm.at[pl.ds((sc * _GS + j) * _CHUNK, _CHUNK)],
                                rows_v.at[pl.ds(j * _CHUNK, _CHUNK)], semA)

                for j in range(_GS):
                    @pl.when(sc * _GS + j < nchunk)
                    def _(j=j):
                        pltpu.make_async_copy(
                            r0_hbm.at[pl.ds((sc * _GS + j) * _CHUNK, _CHUNK)],
                            rows_v.at[pl.ds(j * _CHUNK, _CHUNK)], semA).wait()
                for j in range(_GS):
                    @pl.when(sc * _GS + j < nchunk)
                    def _(j=j):
                        pltpu.async_copy(
                            rows_v.at[pl.ds(j * _CHUNK, _CHUNK)],
                            acc_sh.at[idx_v.at[j]], semB, add=True)
                for j in range(_GS):
                    @pl.when(sc * _GS + j < nchunk)
                    def _(j=j):
                        pltpu.make_async_copy(
                            rows_v.at[pl.ds(j * _CHUNK, _CHUNK)],
                            acc_sh.at[idx_v.at[j]], semB).wait()

            return carry

        lax.fori_loop(0, per_t, body, 0)
        plsc.subcore_barrier()

        @pl.when(jnp.logical_and(s < _NS - 1, c == 0))
        def _():
            pltpu.sync_copy(acc_sh.at[pl.ds(rbase, rpt)],
                            out0_hbm.at[pl.ds(rbase, rpt)])

        @pl.when(jnp.logical_and(s == _NS - 1, c == 0))
        def _():
            pltpu.sync_copy(acc_sh.at[pl.ds((_NS - 1) * rpt, tail)],
                            out0_hbm.at[pl.ds((_NS - 1) * rpt, tail)])

        @pl.when(jnp.logical_and(s < _NS - 1, c == 1))
        def _():
            pltpu.sync_copy(acc_sh.at[pl.ds(rbase, rpt)],
                            out1_hbm.at[pl.ds(rbase, rpt)])

        @pl.when(jnp.logical_and(s == _NS - 1, c == 1))
        def _():
            pltpu.sync_copy(acc_sh.at[pl.ds((_NS - 1) * rpt, tail)],
                            out1_hbm.at[pl.ds((_NS - 1) * rpt, tail)])

    return k(rows0, rows1, idx0_2, idx1_2, zeros)


def kernel(x, edge_index, edge_attr, n_elements, bc_disp, bc_rot, params):
    n = x.shape[0]
    e2 = edge_attr.shape[0]
    e = e2 // 2

    idx_src = edge_index[0, :e].astype(jnp.int32)
    idx_dst = edge_index[1, :e].astype(jnp.int32)
    zeros = jnp.zeros((n, H), jnp.float32)
    nchunk = e // _CHUNK

    def _pad3(idx, g):
        nsc = -(-nchunk // g)
        return jnp.pad(idx.reshape(nchunk, _CHUNK),
                       ((0, nsc * g - nchunk), (0, 0))).reshape(nsc, g, _CHUNK)

    idx_dst_g = _pad3(idx_dst, _GP)
    idx_src_g = _pad3(idx_src, _GP)
    idx_dst_s = _pad3(idx_dst, _GS)
    idx_src_s = _pad3(idx_src, _GS)


    ne = params["node_enc"]
    (nw1, nb1), (nw2, nb2) = ne["layers"]
    ng, nbeta = ne["ln"]
    xp = jnp.pad(x, ((0, 0), (0, 16 - x.shape[1])))
    nw1p = jnp.pad(nw1, ((0, 16 - nw1.shape[0]), (0, 0)))
    h = _enc_call(xp, nw1p, nw2, _aux_pack(nb1, nb2, ng, nbeta),
                  _blk(x.shape[0], NBLK))

    ee = params["edge_enc"]
    (ew1, eb1), (ew2, eb2) = ee["layers"]
    eg, ebeta = ee["ln"]
    eap = jnp.pad(edge_attr, ((0, 0), (0, 8 - edge_attr.shape[1])))
    ew1p = jnp.pad(ew1, ((0, 8 - ew1.shape[0]), (0, 0)))
    eaux = _aux_pack(eb1, eb2, eg, ebeta)
    e_fwd0 = _enc_call(eap[:e], ew1p, ew2, eaux, _blk(e, EBLK))
    e_bwd0 = _enc_call(eap[e:], ew1p, ew2, eaux, _blk(e, EBLK))

    e_fwd = e_fwd0
    n_layers = len(params["mp"])
    for li, layer in enumerate(params["mp"]):
        (w1, b1), (w2, b2) = layer["edge_mlp"]["layers"]
        g, beta = layer["edge_mlp"]["ln"]
        w1a, w1b, w1c = w1[:H], w1[H:2 * H], w1[2 * H:]
        eaux = _aux_pack(b1, b2, g, beta)

        p, q = _pq_call(h, w1b, w1c)
        r = _sc_gather(p, q, idx_dst_g, idx_src_g, e)

        if li == n_layers - 1:
            msg, e_fwd_new, e_bwd_f = _edge_last_call(
                e_fwd, r, e_fwd0, e_bwd0, w1a, w2, eaux)
        else:
            msg, e_fwd_new = _edge_call(e_fwd, r, w1a, w2, eaux)
        e_fwd = e_fwd_new

        a0, a1 = _sc_scatter(n, msg, idx_dst_s, msg, idx_src_s, zeros, e)

        (v1, c1), (v2, c2) = layer["node_mlp"]["layers"]
        vg, vbeta = layer["node_mlp"]["ln"]
        v1a, v1b = v1[:H], v1[H:]
        h = _node_call(h, a0, a1, v1a, v1b, v2,
                       _aux_pack(c1, c2, vg, vbeta))

    i0, i1 = _sc_scatter(n, e_fwd, idx_dst_s, e_bwd_f, idx_src_s, zeros, e)

    fg, fbeta = params["final_ln"]
    lnaux = jnp.stack([fg, fbeta])
    lnaux = jnp.concatenate([lnaux, jnp.zeros((6, 2 * H), jnp.float32)])

    (aux_w1, aux_b1), (aux_w2, aux_b2) = params["dec_ux"]["layers"]
    (auz_w1, auz_b1), (auz_w2, auz_b2) = params["dec_uz"]["layers"]
    (ath_w1, ath_b1), (ath_w2, ath_b2) = params["dec_th"]["layers"]
    acomb = jnp.concatenate([aux_w1, auz_w1, ath_w1], axis=1)
    abias = jnp.concatenate([aux_b1, auz_b1, ath_b1])
    aaux = _aux_pack(abias)
    bblk = jnp.zeros((3 * H, H), jnp.float32)
    bblk = bblk.at[0:H, 0:1].set(aux_w2)
    bblk = bblk.at[H:2 * H, 1:2].set(auz_w2)
    bblk = bblk.at[2 * H:3 * H, 2:3].set(ath_w2)
    bbias = jnp.zeros((H,), jnp.float32)
    bbias = bbias.at[0].set(aux_b2[0]).at[1].set(auz_b2[0]).at[2].set(ath_b2[0])
    baux = _aux_pack(bbias)

    pred = _final_call(h, i0, i1, bc_disp, bc_rot,
                       lnaux, acomb, aaux, bblk, baux)
    return pred[:, :3]
